# baseline (speedup 1.0000x reference)
import functools

import jax
import jax.numpy as jnp
from jax.experimental import pallas as pl
from jax.experimental.pallas import tpu as pltpu
from jax.experimental.pallas import tpu_sc as plsc

_B = 16384
_D = 128
_GW = 128
_BS = 2048


def _sc_gather(user_idx, item_idx, user_table, item_table):
    mesh = plsc.VectorSubcoreMesh(core_axis_name="core",
                                  subcore_axis_name="subcore")

    @functools.partial(
        pl.kernel,
        out_type=(jax.ShapeDtypeStruct((_B, _D), jnp.float32),
                  jax.ShapeDtypeStruct((_B, _D), jnp.float32)),
        mesh=mesh,
    )
    def gather_kernel(ut_hbm, it_hbm, ui_hbm, ii_hbm, eu_hbm, ev_hbm):
        def body(ui_vmem, ii_vmem, eu_vmem, ev_vmem):
            pltpu.sync_copy(ut_hbm.at[ui_vmem.at[0]], eu_vmem)
            pltpu.sync_copy(it_hbm.at[ii_vmem.at[0]], ev_vmem)

        pltpu.emit_pipeline(
            body,
            grid=(_B // _GW,),
            in_specs=[pl.BlockSpec((1, _GW), lambda i: (0, i)),
                      pl.BlockSpec((1, _GW), lambda i: (0, i))],
            out_specs=[pl.BlockSpec((_GW, _D), lambda i: (i, 0)),
                       pl.BlockSpec((_GW, _D), lambda i: (i, 0))],
            core_axis_name=("core", "subcore"),
            dimension_semantics=(pltpu.PARALLEL,),
        )(ui_hbm, ii_hbm, eu_hbm, ev_hbm)

    return gather_kernel(user_table, item_table, user_idx, item_idx)


def _mlp_body(eu_ref, ev_ref, w1u_ref, w1v_ref, b1_ref, w2_ref, b2_ref,
              w3_ref, b3_ref, w4_ref, b4_ref, wp_ref, bp_ref, out_ref):
    bf = jnp.bfloat16

    def dot(a, w_ref):
        return jnp.dot(a.astype(bf), w_ref[...].astype(bf),
                       preferred_element_type=jnp.float32)

    x = dot(eu_ref[...], w1u_ref) + dot(ev_ref[...], w1v_ref)
    x = jnp.maximum(x + b1_ref[...], 0.0)
    x = jnp.maximum(dot(x, w2_ref) + b2_ref[...], 0.0)
    x = jnp.maximum(dot(x, w3_ref) + b3_ref[...], 0.0)
    x = jnp.maximum(dot(x, w4_ref) + b4_ref[...], 0.0)
    out_ref[...] = dot(x, wp_ref) + bp_ref[...]


def _mlp(eu, ev, w1u, w1v, b1, w2, b2, w3, b3, w4, b4, wp, bp):
    def _full(a):
        return pl.BlockSpec(a.shape, lambda i: (0,) * a.ndim)

    return pl.pallas_call(
        _mlp_body,
        grid=(_B // _BS,),
        in_specs=[
            pl.BlockSpec((_BS, _D), lambda i: (i, 0)),
            pl.BlockSpec((_BS, _D), lambda i: (i, 0)),
            _full(w1u), _full(w1v), _full(b1), _full(w2), _full(b2),
            _full(w3), _full(b3), _full(w4), _full(b4), _full(wp), _full(bp),
        ],
        out_specs=pl.BlockSpec((_BS, 1), lambda i: (i, 0)),
        out_shape=jax.ShapeDtypeStruct((_B, 1), jnp.float32),
        compiler_params=pltpu.CompilerParams(
            dimension_semantics=("arbitrary",)),
    )(eu, ev, w1u, w1v, b1, w2, b2, w3, b3, w4, b4, wp, bp)


def kernel(user, item, user_table, item_table,
           W1, b1, W2, b2, W3, b3, W4, b4, Wp, bp):
    ui = user.astype(jnp.int32).reshape(1, _B)
    ii = item.astype(jnp.int32).reshape(1, _B)
    eu, ev = _sc_gather(ui, ii, user_table, item_table)
    out = _mlp(eu, ev, W1[:_D], W1[_D:], b1.reshape(1, -1),
               W2, b2.reshape(1, -1), W3, b3.reshape(1, -1),
               W4, b4.reshape(1, -1), Wp, bp.reshape(1, 1))
    return out.reshape(-1)


# speedup vs baseline: 1.0979x; 1.0979x over previous
import functools

import jax
import jax.numpy as jnp
from jax.experimental import pallas as pl
from jax.experimental.pallas import tpu as pltpu
from jax.experimental.pallas import tpu_sc as plsc

_B = 16384
_D = 128
_GW = 128
_BS = 2048


def _sc_gather(user_idx, item_idx, user_table, item_table):
    mesh = plsc.VectorSubcoreMesh(core_axis_name="core",
                                  subcore_axis_name="subcore")

    @functools.partial(
        pl.kernel,
        out_type=(jax.ShapeDtypeStruct((_B, _D), jnp.float32),
                  jax.ShapeDtypeStruct((_B, _D), jnp.float32)),
        mesh=mesh,
    )
    def gather_kernel(ut_hbm, it_hbm, ui_hbm, ii_hbm, eu_hbm, ev_hbm):
        def body(ui_vmem, ii_vmem, eu_vmem, ev_vmem):
            def inner(s1, s2):
                c1 = pltpu.async_copy(ut_hbm.at[ui_vmem.at[0]], eu_vmem, s1)
                c2 = pltpu.async_copy(it_hbm.at[ii_vmem.at[0]], ev_vmem, s2)
                c1.wait()
                c2.wait()

            pl.run_scoped(inner, pltpu.SemaphoreType.DMA,
                          pltpu.SemaphoreType.DMA)

        pltpu.emit_pipeline(
            body,
            grid=(_B // _GW,),
            in_specs=[pl.BlockSpec((1, _GW), lambda i: (0, i)),
                      pl.BlockSpec((1, _GW), lambda i: (0, i))],
            out_specs=[pl.BlockSpec((_GW, _D), lambda i: (i, 0)),
                       pl.BlockSpec((_GW, _D), lambda i: (i, 0))],
            core_axis_name=("core", "subcore"),
            dimension_semantics=(pltpu.PARALLEL,),
        )(ui_hbm, ii_hbm, eu_hbm, ev_hbm)

    return gather_kernel(user_table, item_table, user_idx, item_idx)


def _mlp_body(eu_ref, ev_ref, w1_ref, b1_ref, w2_ref, b2_ref,
              w3_ref, b3_ref, w4_ref, b4_ref, wp_ref, bp_ref, out_ref):
    bf = jnp.bfloat16

    def dot(a, w):
        return jnp.dot(a.astype(bf), w.astype(bf),
                       preferred_element_type=jnp.float32)

    x = dot(eu_ref[...], w1_ref[0:_D, :]) + dot(ev_ref[...], w1_ref[_D:, :])
    x = jnp.maximum(x + b1_ref[...], 0.0)
    x = jnp.maximum(dot(x, w2_ref[...]) + b2_ref[...], 0.0)
    x = jnp.maximum(dot(x, w3_ref[...]) + b3_ref[...], 0.0)
    x = jnp.maximum(dot(x, w4_ref[...]) + b4_ref[...], 0.0)
    out_ref[...] = (dot(x, wp_ref[...]) + bp_ref[...]).reshape(-1)


def _mlp(eu, ev, w1, b1, w2, b2, w3, b3, w4, b4, wp, bp):
    def _full(a):
        return pl.BlockSpec(a.shape, lambda i: (0,) * a.ndim)

    return pl.pallas_call(
        _mlp_body,
        grid=(_B // _BS,),
        in_specs=[
            pl.BlockSpec((_BS, _D), lambda i: (i, 0)),
            pl.BlockSpec((_BS, _D), lambda i: (i, 0)),
            _full(w1), _full(b1), _full(w2), _full(b2),
            _full(w3), _full(b3), _full(w4), _full(b4), _full(wp), _full(bp),
        ],
        out_specs=pl.BlockSpec((_BS,), lambda i: (i,)),
        out_shape=jax.ShapeDtypeStruct((_B,), jnp.float32),
        compiler_params=pltpu.CompilerParams(
            dimension_semantics=("arbitrary",)),
    )(eu, ev, w1, b1, w2, b2, w3, b3, w4, b4, wp, bp)


def kernel(user, item, user_table, item_table,
           W1, b1, W2, b2, W3, b3, W4, b4, Wp, bp):
    ui = user.astype(jnp.int32).reshape(1, _B)
    ii = item.astype(jnp.int32).reshape(1, _B)
    eu, ev = _sc_gather(ui, ii, user_table, item_table)
    return _mlp(eu, ev, W1, b1, W2, b2, W3, b3, W4, b4, Wp, bp)


# speedup vs baseline: 1.1034x; 1.0050x over previous
import functools

import jax
import jax.numpy as jnp
from jax.experimental import pallas as pl
from jax.experimental.pallas import tpu as pltpu
from jax.experimental.pallas import tpu_sc as plsc

_B = 16384
_D = 128
_GW = 128
_BS = 4096


def _sc_gather(user_idx, item_idx, user_table, item_table):
    mesh = plsc.VectorSubcoreMesh(core_axis_name="core",
                                  subcore_axis_name="subcore")

    @functools.partial(
        pl.kernel,
        out_type=(jax.ShapeDtypeStruct((_B, _D), jnp.float32),
                  jax.ShapeDtypeStruct((_B, _D), jnp.float32)),
        mesh=mesh,
    )
    def gather_kernel(ut_hbm, it_hbm, ui_hbm, ii_hbm, eu_hbm, ev_hbm):
        def body(ui_vmem, ii_vmem, eu_vmem, ev_vmem):
            def inner(s1, s2):
                c1 = pltpu.async_copy(ut_hbm.at[ui_vmem], eu_vmem, s1)
                c2 = pltpu.async_copy(it_hbm.at[ii_vmem], ev_vmem, s2)
                c1.wait()
                c2.wait()

            pl.run_scoped(inner, pltpu.SemaphoreType.DMA,
                          pltpu.SemaphoreType.DMA)

        pltpu.emit_pipeline(
            body,
            grid=(_B // _GW,),
            in_specs=[pl.BlockSpec((_GW,), lambda i: (i,)),
                      pl.BlockSpec((_GW,), lambda i: (i,))],
            out_specs=[pl.BlockSpec((_GW, _D), lambda i: (i, 0)),
                       pl.BlockSpec((_GW, _D), lambda i: (i, 0))],
            core_axis_name=("core", "subcore"),
            dimension_semantics=(pltpu.PARALLEL,),
        )(ui_hbm, ii_hbm, eu_hbm, ev_hbm)

    return gather_kernel(user_table, item_table, user_idx, item_idx)


def _mlp_body(eu_ref, ev_ref, w1_ref, b1_ref, w2_ref, b2_ref,
              w3_ref, b3_ref, w4_ref, b4_ref, wp_ref, bp_ref, out_ref):
    bf = jnp.bfloat16

    def dot(a, w):
        return jnp.dot(a.astype(bf), w.astype(bf),
                       preferred_element_type=jnp.float32)

    x = dot(jnp.concatenate([eu_ref[...], ev_ref[...]], axis=-1), w1_ref[...])
    x = jnp.maximum(x + b1_ref[...], 0.0)
    x = jnp.maximum(dot(x, w2_ref[...]) + b2_ref[...], 0.0)
    x = jnp.maximum(dot(x, w3_ref[...]) + b3_ref[...], 0.0)
    x = jnp.maximum(dot(x, w4_ref[...]) + b4_ref[...], 0.0)
    out_ref[...] = (dot(x, wp_ref[...]) + bp_ref[...]).reshape(-1)


def _mlp(eu, ev, w1, b1, w2, b2, w3, b3, w4, b4, wp, bp):
    def _full(a):
        return pl.BlockSpec(a.shape, lambda i: (0,) * a.ndim)

    return pl.pallas_call(
        _mlp_body,
        grid=(_B // _BS,),
        in_specs=[
            pl.BlockSpec((_BS, _D), lambda i: (i, 0)),
            pl.BlockSpec((_BS, _D), lambda i: (i, 0)),
            _full(w1), _full(b1), _full(w2), _full(b2),
            _full(w3), _full(b3), _full(w4), _full(b4), _full(wp), _full(bp),
        ],
        out_specs=pl.BlockSpec((_BS,), lambda i: (i,)),
        out_shape=jax.ShapeDtypeStruct((_B,), jnp.float32),
        compiler_params=pltpu.CompilerParams(
            dimension_semantics=("arbitrary",)),
    )(eu, ev, w1, b1, w2, b2, w3, b3, w4, b4, wp, bp)


def kernel(user, item, user_table, item_table,
           W1, b1, W2, b2, W3, b3, W4, b4, Wp, bp):
    eu, ev = _sc_gather(user.astype(jnp.int32), item.astype(jnp.int32),
                        user_table, item_table)
    return _mlp(eu, ev, W1, b1, W2, b2, W3, b3, W4, b4, Wp, bp)


# speedup vs baseline: 1.1171x; 1.0124x over previous
import functools

import jax
import jax.numpy as jnp
from jax.experimental import pallas as pl
from jax.experimental.pallas import tpu as pltpu
from jax.experimental.pallas import tpu_sc as plsc

_B = 16384
_D = 128
_GW = 128
_BS = 4096


def _sc_gather(user_idx, item_idx, user_table, item_table):
    mesh = plsc.VectorSubcoreMesh(core_axis_name="core",
                                  subcore_axis_name="subcore")

    @functools.partial(
        pl.kernel,
        out_type=(jax.ShapeDtypeStruct((_B, _D), jnp.float32),
                  jax.ShapeDtypeStruct((_B, _D), jnp.float32)),
        mesh=mesh,
    )
    def gather_kernel(ut_hbm, it_hbm, ui_hbm, ii_hbm, eu_hbm, ev_hbm):
        def body(ui_vmem, ii_vmem, eu_vmem, ev_vmem):
            def inner(s1, s2):
                c1 = pltpu.async_copy(ut_hbm.at[ui_vmem], eu_vmem, s1)
                c2 = pltpu.async_copy(it_hbm.at[ii_vmem], ev_vmem, s2)
                c1.wait()
                c2.wait()

            pl.run_scoped(inner, pltpu.SemaphoreType.DMA,
                          pltpu.SemaphoreType.DMA)

        pltpu.emit_pipeline(
            body,
            grid=(_B // _GW,),
            in_specs=[pl.BlockSpec((_GW,), lambda i: (i,)),
                      pl.BlockSpec((_GW,), lambda i: (i,))],
            out_specs=[pl.BlockSpec((_GW, _D), lambda i: (i, 0)),
                       pl.BlockSpec((_GW, _D), lambda i: (i, 0))],
            core_axis_name=("core", "subcore"),
            dimension_semantics=(pltpu.PARALLEL,),
        )(ui_hbm, ii_hbm, eu_hbm, ev_hbm)

    return gather_kernel(user_table, item_table, user_idx, item_idx)


def _mlp_body(eu_ref, ev_ref, w1_ref, b1_ref, w2_ref, b2_ref,
              w3_ref, b3_ref, w4_ref, b4_ref, wp_ref, bp_ref, out_ref):
    def dot(a, w):
        return jnp.dot(a, w, preferred_element_type=jnp.float32)

    x = dot(eu_ref[...], w1_ref[0:_D, :]) + dot(ev_ref[...], w1_ref[_D:, :])
    x = jnp.maximum(x + b1_ref[...], 0.0)
    x = jnp.maximum(dot(x, w2_ref[...]) + b2_ref[...], 0.0)
    x = jnp.maximum(dot(x, w3_ref[...]) + b3_ref[...], 0.0)
    x = jnp.maximum(dot(x, w4_ref[...]) + b4_ref[...], 0.0)
    out_ref[...] = (dot(x, wp_ref[...]) + bp_ref[...]).reshape(-1)


def _mlp(eu, ev, w1, b1, w2, b2, w3, b3, w4, b4, wp, bp):
    def _full(a):
        return pl.BlockSpec(a.shape, lambda i: (0,) * a.ndim)

    return pl.pallas_call(
        _mlp_body,
        grid=(_B // _BS,),
        in_specs=[
            pl.BlockSpec((_BS, _D), lambda i: (i, 0)),
            pl.BlockSpec((_BS, _D), lambda i: (i, 0)),
            _full(w1), _full(b1), _full(w2), _full(b2),
            _full(w3), _full(b3), _full(w4), _full(b4), _full(wp), _full(bp),
        ],
        out_specs=pl.BlockSpec((_BS,), lambda i: (i,)),
        out_shape=jax.ShapeDtypeStruct((_B,), jnp.float32),
        compiler_params=pltpu.CompilerParams(
            dimension_semantics=("arbitrary",)),
    )(eu, ev, w1, b1, w2, b2, w3, b3, w4, b4, wp, bp)


def kernel(user, item, user_table, item_table,
           W1, b1, W2, b2, W3, b3, W4, b4, Wp, bp):
    eu, ev = _sc_gather(user.astype(jnp.int32), item.astype(jnp.int32),
                        user_table, item_table)
    return _mlp(eu, ev, W1, b1, W2, b2, W3, b3, W4, b4, Wp, bp)
